# Initial kernel scaffold; baseline (speedup 1.0000x reference)
import functools

import jax
import jax.numpy as jnp
from jax import lax
from jax.experimental import pallas as pl
from jax.experimental.pallas import tpu as pltpu
from jax.experimental.pallas import tpu_sc as plsc

N = 10000
D = 128
E = 320000

NC = 2
NS = 16
NW = NC * NS
L = 16

K = 128
NPAD = 10112
EPT = 10112
CHUNKS = EPT // K
EPAD = EPT * NW
RPT = NPAD // NS

_BLK = 400


def _prep_body(x_ref, w_ref, m1_ref, m2_ref, g_ref, b_ref, mu_ref, var_ref,
               mapped_ref, s1_ref, s2_ref):
    eps = 1e-3
    scale = g_ref[...] * lax.rsqrt(var_ref[...] + eps)
    xh = (x_ref[...] - mu_ref[...]) * scale + b_ref[...]
    dot = functools.partial(jnp.dot, preferred_element_type=jnp.float32,
                            precision=lax.Precision.HIGHEST)
    mapped_ref[...] = dot(xh, w_ref[...])
    p1 = dot(xh, m1_ref[...])
    s1_ref[...] = jnp.tanh(jnp.sum(p1 * xh, axis=1, keepdims=True))
    p2 = dot(xh, m2_ref[...])
    s2_ref[...] = jnp.tanh(jnp.sum(p2 * xh, axis=1, keepdims=True))


def _prep(x, W, M1, M2, gamma, beta, mu, var):
    full = pl.BlockSpec((D, D), lambda i: (0, 0))
    vec = pl.BlockSpec((1, D), lambda i: (0, 0))
    return pl.pallas_call(
        _prep_body,
        grid=(N // _BLK,),
        in_specs=[pl.BlockSpec((_BLK, D), lambda i: (i, 0)),
                  full, full, full, vec, vec, vec, vec],
        out_specs=[pl.BlockSpec((_BLK, D), lambda i: (i, 0)),
                   pl.BlockSpec((_BLK, 1), lambda i: (i, 0)),
                   pl.BlockSpec((_BLK, 1), lambda i: (i, 0))],
        out_shape=[jax.ShapeDtypeStruct((N, D), jnp.float32),
                   jax.ShapeDtypeStruct((N, 1), jnp.float32),
                   jax.ShapeDtypeStruct((N, 1), jnp.float32)],
    )(x, W, M1, M2, gamma.reshape(1, D), beta.reshape(1, D),
      mu.reshape(1, D), var.reshape(1, D))


def _edge_body(mapped_hbm, s1_hbm, s2_hbm, row_hbm, col_hbm, a_hbm,
               acc_hbm, den_hbm,
               s1_v, s2_v, row_v, col_v, a_v, ex_v, gbuf, zden, acc_sp,
               den_sp, sem):
    cid = lax.axis_index("c")
    sid = lax.axis_index("s")
    wid = cid * NS + sid

    zero16 = jnp.zeros((L,), jnp.float32)

    pltpu.sync_copy(s1_hbm, s1_v.at[pl.ds(0, N)])
    pltpu.sync_copy(s2_hbm, s2_v.at[pl.ds(0, N)])
    for t in range((NPAD - N) // L):
        s1_v[pl.ds(N + t * L, L)] = zero16
        s2_v[pl.ds(N + t * L, L)] = zero16

    @pl.loop(0, K)
    def _zg(i):
        for k in range(D // L):
            gbuf[i, pl.ds(k * L, L)] = zero16

    for t in range(640 // L):
        zden[pl.ds(t * L, L)] = zero16

    rbase = sid * RPT
    pltpu.sync_copy(gbuf, acc_sp.at[pl.ds(rbase, K)])
    pltpu.sync_copy(gbuf, acc_sp.at[pl.ds(rbase + K, K)])
    pltpu.sync_copy(gbuf, acc_sp.at[pl.ds(rbase + 2 * K, K)])
    pltpu.sync_copy(gbuf, acc_sp.at[pl.ds(rbase + 3 * K, K)])
    pltpu.sync_copy(gbuf.at[pl.ds(0, RPT - 4 * K)],
                    acc_sp.at[pl.ds(rbase + 4 * K, RPT - 4 * K)])
    pltpu.sync_copy(zden.at[pl.ds(0, RPT)], den_sp.at[pl.ds(rbase, RPT)])

    plsc.subcore_barrier()

    ebase = wid * EPT

    @pl.loop(0, CHUNKS)
    def _chunk(cc):
        off = ebase + cc * K
        pltpu.sync_copy(row_hbm.at[pl.ds(off, K)], row_v)
        pltpu.sync_copy(col_hbm.at[pl.ds(off, K)], col_v)
        pltpu.sync_copy(a_hbm.at[pl.ds(off, K)], a_v)
        pltpu.async_copy(mapped_hbm.at[col_v], gbuf, sem).wait()

        for j in range(K // L):
            r16 = row_v[pl.ds(j * L, L)]
            c16 = col_v[pl.ds(j * L, L)]
            av = a_v[pl.ds(j * L, L)]
            g1 = plsc.load_gather(s1_v, [r16])
            g2 = plsc.load_gather(s2_v, [c16])
            ev = av * g1 + av * g2
            ev = jnp.where(ev >= 0.0, ev, 0.2 * ev)
            ex_v[pl.ds(j * L, L)] = jnp.exp(ev)

        @pl.loop(0, K)
        def _scale(i):
            e = ex_v[i]
            for k in range(D // L):
                gbuf[i, pl.ds(k * L, L)] = gbuf[i, pl.ds(k * L, L)] * e

        pltpu.sync_copy(gbuf, acc_sp.at[row_v], add=True)
        pltpu.sync_copy(ex_v, den_sp.at[row_v], add=True)

    plsc.subcore_barrier()

    pltpu.sync_copy(acc_sp.at[pl.ds(rbase, RPT)],
                    acc_hbm.at[cid, pl.ds(rbase, RPT)])
    pltpu.sync_copy(den_sp.at[pl.ds(rbase, RPT)],
                    den_hbm.at[cid, pl.ds(rbase, RPT)])


def _edge(mapped, s1, s2, rowp, colp, ap):
    mesh = plsc.VectorSubcoreMesh(core_axis_name="c", subcore_axis_name="s")
    f = pl.kernel(
        _edge_body,
        out_type=(jax.ShapeDtypeStruct((NC, NPAD, D), jnp.float32),
                  jax.ShapeDtypeStruct((NC, NPAD), jnp.float32)),
        mesh=mesh,
        scratch_types=[
            pltpu.VMEM((NPAD,), jnp.float32),
            pltpu.VMEM((NPAD,), jnp.float32),
            pltpu.VMEM((K,), jnp.int32),
            pltpu.VMEM((K,), jnp.int32),
            pltpu.VMEM((K,), jnp.float32),
            pltpu.VMEM((K,), jnp.float32),
            pltpu.VMEM((K, D), jnp.float32),
            pltpu.VMEM((640,), jnp.float32),
            pltpu.VMEM_SHARED((NPAD, D), jnp.float32),
            pltpu.VMEM_SHARED((NPAD,), jnp.float32),
            pltpu.SemaphoreType.DMA,
        ],
    )
    return f(mapped, s1, s2, rowp, colp, ap)


def _final_body(acc_ref, den_ref, out_ref):
    acc = acc_ref[0] + acc_ref[1]
    den = den_ref[0, :, :] + den_ref[1, :, :]
    out_ref[...] = acc * (1.0 / jnp.maximum(den, 1e-30))


def _final(acc, den):
    return pl.pallas_call(
        _final_body,
        grid=(N // _BLK,),
        in_specs=[pl.BlockSpec((NC, _BLK, D), lambda i: (0, i, 0)),
                  pl.BlockSpec((NC, _BLK, 1), lambda i: (0, i, 0))],
        out_specs=pl.BlockSpec((_BLK, D), lambda i: (i, 0)),
        out_shape=jax.ShapeDtypeStruct((N, D), jnp.float32),
    )(acc, den)


def kernel(x, edge_index, a_vals, W, M1, M2, gamma, beta, moving_mean,
           moving_var):
    mapped, s1, s2 = _prep(x, W, M1, M2, gamma, beta, moving_mean, moving_var)
    row = edge_index[0]
    col = edge_index[1]
    pad = EPAD - E
    rowp = jnp.concatenate([row, jnp.full((pad,), N, jnp.int32)])
    colp = jnp.concatenate([col, jnp.zeros((pad,), jnp.int32)])
    ap = jnp.concatenate([a_vals, jnp.zeros((pad,), jnp.float32)])
    acc, den = _edge(mapped, s1.reshape(N), s2.reshape(N), rowp, colp, ap)
    return _final(acc, den.reshape(NC, NPAD, 1))


# speedup vs baseline: 14.6007x; 14.6007x over previous
import dataclasses
import functools

import jax
import jax.numpy as jnp
from jax import lax
from jax.experimental import pallas as pl
from jax.experimental.pallas import tpu as pltpu
from jax.experimental.pallas import tpu_sc as plsc

N = 10000
D = 128
E = 320000

NC = 2
NS = 16
NW = NC * NS
L = 16

K = 128
NPAD = 10112
EPT = 10112
CHUNKS = EPT // K
EPAD = EPT * NW
RPT = NPAD // NS

_BLK = 400


def _prep_body(x_ref, w_ref, m1_ref, m2_ref, g_ref, b_ref, mu_ref, var_ref,
               mapped_ref, s1_ref, s2_ref):
    eps = 1e-3
    scale = g_ref[...] * lax.rsqrt(var_ref[...] + eps)
    xh = (x_ref[...] - mu_ref[...]) * scale + b_ref[...]
    dot = functools.partial(jnp.dot, preferred_element_type=jnp.float32,
                            precision=lax.Precision.HIGHEST)
    mapped_ref[...] = dot(xh, w_ref[...])
    p1 = dot(xh, m1_ref[...])
    s1_ref[...] = jnp.tanh(jnp.sum(p1 * xh, axis=1, keepdims=True))
    p2 = dot(xh, m2_ref[...])
    s2_ref[...] = jnp.tanh(jnp.sum(p2 * xh, axis=1, keepdims=True))


def _prep(x, W, M1, M2, gamma, beta, mu, var):
    full = pl.BlockSpec((D, D), lambda i: (0, 0))
    vec = pl.BlockSpec((1, D), lambda i: (0, 0))
    return pl.pallas_call(
        _prep_body,
        grid=(N // _BLK,),
        in_specs=[pl.BlockSpec((_BLK, D), lambda i: (i, 0)),
                  full, full, full, vec, vec, vec, vec],
        out_specs=[pl.BlockSpec((_BLK, D), lambda i: (i, 0)),
                   pl.BlockSpec((_BLK, 1), lambda i: (i, 0)),
                   pl.BlockSpec((_BLK, 1), lambda i: (i, 0))],
        out_shape=[jax.ShapeDtypeStruct((N, D), jnp.float32),
                   jax.ShapeDtypeStruct((N, 1), jnp.float32),
                   jax.ShapeDtypeStruct((N, 1), jnp.float32)],
    )(x, W, M1, M2, gamma.reshape(1, D), beta.reshape(1, D),
      mu.reshape(1, D), var.reshape(1, D))


def _edge_body(mapped_hbm, s1_hbm, s2_hbm, row_hbm, col_hbm, a_hbm,
               acc_hbm, den_hbm,
               s1_v, s2_v, row_v, col_v, a_v, ex_v, gbuf, zden, acc_sp,
               den_sp, sem):
    cid = lax.axis_index("c")
    sid = lax.axis_index("s")
    wid = cid * NS + sid

    zero16 = jnp.zeros((L,), jnp.float32)

    pltpu.sync_copy(s1_hbm, s1_v.at[pl.ds(0, N)])
    pltpu.sync_copy(s2_hbm, s2_v.at[pl.ds(0, N)])
    for t in range((NPAD - N) // L):
        s1_v[pl.ds(N + t * L, L)] = zero16
        s2_v[pl.ds(N + t * L, L)] = zero16

    @pl.loop(0, K)
    def _zg(i):
        for k in range(D // L):
            gbuf[i, pl.ds(k * L, L)] = zero16

    for t in range(640 // L):
        zden[pl.ds(t * L, L)] = zero16

    rbase = sid * RPT
    pltpu.sync_copy(gbuf, acc_sp.at[pl.ds(rbase, K)])
    pltpu.sync_copy(gbuf, acc_sp.at[pl.ds(rbase + K, K)])
    pltpu.sync_copy(gbuf, acc_sp.at[pl.ds(rbase + 2 * K, K)])
    pltpu.sync_copy(gbuf, acc_sp.at[pl.ds(rbase + 3 * K, K)])
    pltpu.sync_copy(gbuf.at[pl.ds(0, RPT - 4 * K)],
                    acc_sp.at[pl.ds(rbase + 4 * K, RPT - 4 * K)])
    pltpu.sync_copy(zden.at[pl.ds(0, RPT)], den_sp.at[pl.ds(rbase, RPT)])

    plsc.subcore_barrier()

    ebase = wid * EPT

    @pl.loop(0, CHUNKS)
    def _chunk(cc):
        off = ebase + cc * K
        pltpu.sync_copy(row_hbm.at[pl.ds(off, K)], row_v)
        pltpu.sync_copy(col_hbm.at[pl.ds(off, K)], col_v)
        pltpu.sync_copy(a_hbm.at[pl.ds(off, K)], a_v)
        pltpu.async_copy(mapped_hbm.at[col_v], gbuf, sem).wait()

        for j in range(K // L):
            r16 = row_v[pl.ds(j * L, L)]
            c16 = col_v[pl.ds(j * L, L)]
            av = a_v[pl.ds(j * L, L)]
            g1 = plsc.load_gather(s1_v, [r16])
            g2 = plsc.load_gather(s2_v, [c16])
            ev = av * g1 + av * g2
            ev = jnp.where(ev >= 0.0, ev, 0.2 * ev)
            ex_v[pl.ds(j * L, L)] = jnp.exp(ev)

        @pl.loop(0, K // L)
        def _scale(g):
            base = pl.multiple_of(g * L, L)
            exg = ex_v[pl.ds(base, L)]
            for l in range(L):
                i = base + l
                e = exg[l]
                for k in range(D // L):
                    gbuf[i, pl.ds(k * L, L)] = gbuf[i, pl.ds(k * L, L)] * e

        pltpu.sync_copy(gbuf, acc_sp.at[row_v], add=True)
        pltpu.sync_copy(ex_v, den_sp.at[row_v], add=True)

    plsc.subcore_barrier()

    pltpu.sync_copy(acc_sp.at[pl.ds(rbase, RPT)],
                    acc_hbm.at[cid, pl.ds(rbase, RPT)])
    pltpu.sync_copy(den_sp.at[pl.ds(rbase, RPT)], zden.at[pl.ds(0, RPT)])
    pltpu.sync_copy(zden.at[pl.ds(0, RPT)],
                    den_hbm.at[pl.ds(cid * NPAD + rbase, RPT)])


def _edge(mapped, s1, s2, rowp, colp, ap):
    mesh = plsc.VectorSubcoreMesh(core_axis_name="c", subcore_axis_name="s")
    cp = pltpu.CompilerParams()
    if "needs_layout_passes" in pltpu.CompilerParams.__dataclass_fields__:
        cp = dataclasses.replace(cp, needs_layout_passes=False)
    f = pl.kernel(
        _edge_body,
        out_type=(jax.ShapeDtypeStruct((NC, NPAD, D), jnp.float32),
                  jax.ShapeDtypeStruct((NC * NPAD,), jnp.float32)),
        mesh=mesh,
        scratch_types=[
            pltpu.VMEM((NPAD,), jnp.float32),
            pltpu.VMEM((NPAD,), jnp.float32),
            pltpu.VMEM((K,), jnp.int32),
            pltpu.VMEM((K,), jnp.int32),
            pltpu.VMEM((K,), jnp.float32),
            pltpu.VMEM((K,), jnp.float32),
            pltpu.VMEM((K, D), jnp.float32),
            pltpu.VMEM((640,), jnp.float32),
            pltpu.VMEM_SHARED((NPAD, D), jnp.float32),
            pltpu.VMEM_SHARED((NPAD,), jnp.float32),
            pltpu.SemaphoreType.DMA,
        ],
        compiler_params=cp,
    )
    return f(mapped, s1, s2, rowp, colp, ap)


def _final_body(acc_ref, den_ref, out_ref):
    acc = acc_ref[0] + acc_ref[1]
    den = den_ref[0, :, :] + den_ref[1, :, :]
    out_ref[...] = acc * (1.0 / jnp.maximum(den, 1e-30))


def _final(acc, den):
    return pl.pallas_call(
        _final_body,
        grid=(N // _BLK,),
        in_specs=[pl.BlockSpec((NC, _BLK, D), lambda i: (0, i, 0)),
                  pl.BlockSpec((NC, _BLK, 1), lambda i: (0, i, 0))],
        out_specs=pl.BlockSpec((_BLK, D), lambda i: (i, 0)),
        out_shape=jax.ShapeDtypeStruct((N, D), jnp.float32),
    )(acc, den)


def kernel(x, edge_index, a_vals, W, M1, M2, gamma, beta, moving_mean,
           moving_var):
    mapped, s1, s2 = _prep(x, W, M1, M2, gamma, beta, moving_mean, moving_var)
    row = edge_index[0]
    col = edge_index[1]
    pad = EPAD - E
    rowp = jnp.concatenate([row, jnp.full((pad,), N, jnp.int32)])
    colp = jnp.concatenate([col, jnp.zeros((pad,), jnp.int32)])
    ap = jnp.concatenate([a_vals, jnp.zeros((pad,), jnp.float32)])
    acc, den = _edge(mapped, s1.reshape(N), s2.reshape(N), rowp, colp, ap)
    return _final(acc, den.reshape(NC, NPAD, 1))
